# baseline (speedup 1.0000x reference)
import functools

import numpy as np

import jax
import jax.numpy as jnp
from jax import lax
from jax.experimental import pallas as pl
from jax.experimental.pallas import tpu as pltpu
from jax.experimental.pallas import tpu_sc as plsc

N_NODES = 10000
D_OUT = 64
NC = 2
NS = 16
L = 16
NW = NC * NS
CHUNK = 128
CHUNKS_PW = 80
EPW = CHUNK * CHUNKS_PW
E_PAD = NW * EPW
N_PAD = 10240
RPW = N_PAD // NS

D_HALF = D_OUT // 2


def _matmul_body(x_ref, w_ref, o_ref):
    z = jnp.dot(x_ref[...], w_ref[...], preferred_element_type=jnp.float32)
    zb = z.astype(jnp.bfloat16)
    lo = lax.bitcast_convert_type(zb[:, :D_HALF], jnp.uint16
                                  ).astype(jnp.uint32)
    hi = lax.bitcast_convert_type(zb[:, D_HALF:], jnp.uint16
                                  ).astype(jnp.uint32)
    o_ref[...] = lax.bitcast_convert_type(lo | (hi << 16), jnp.int32)


def _finish_body(p_ref, b_ref, o_ref):
    z = p_ref[0] + p_ref[1] + b_ref[...]
    m = jnp.max(z, axis=1, keepdims=True)
    zz = z - m
    lse = jnp.log(jnp.sum(jnp.exp(zz), axis=1, keepdims=True))
    o_ref[...] = zz - lse


_mesh = plsc.VectorSubcoreMesh(core_axis_name="c", subcore_axis_name="s")


@functools.partial(
    pl.kernel,
    out_type=jax.ShapeDtypeStruct((NC, N_PAD, D_OUT), jnp.float32),
    mesh=_mesh,
    scratch_types=[
        pltpu.VMEM((CHUNKS_PW, CHUNK), jnp.int32),
        pltpu.VMEM((CHUNKS_PW, CHUNK), jnp.int32),
        pltpu.VMEM((CHUNKS_PW, CHUNK), jnp.float32),
        pltpu.VMEM((CHUNK, D_HALF), jnp.int32),
        pltpu.VMEM((CHUNK, D_HALF), jnp.int32),
        pltpu.VMEM((CHUNK, D_OUT), jnp.float32),
        pltpu.VMEM((CHUNK, D_OUT), jnp.float32),
        pltpu.VMEM_SHARED((N_PAD, D_OUT), jnp.float32),
        pltpu.VMEM_SHARED((N_PAD, D_HALF), jnp.int32),
        pltpu.SemaphoreType.DMA,
        pltpu.SemaphoreType.DMA,
        pltpu.SemaphoreType.DMA,
        pltpu.SemaphoreType.DMA,
    ],
    compiler_params=pltpu.CompilerParams(use_tc_tiling_on_sc=False),
)
def _edge_scatter(support_hbm, src_hbm, dst_hbm, w_hbm, zeros_hbm, out_hbm,
                  src_v, dst_v, w_v, rows0, rows1, msgs0, msgs1, accum,
                  support_s, gsem0, gsem1, ssem0, ssem1):
    c = lax.axis_index("c")
    s = lax.axis_index("s")
    wid = s * NC + c

    pltpu.sync_copy(src_hbm.at[wid], src_v)
    pltpu.sync_copy(dst_hbm.at[wid], dst_v)
    pltpu.sync_copy(w_hbm.at[wid], w_v)
    pltpu.sync_copy(zeros_hbm.at[pl.ds(s * RPW, RPW)],
                    accum.at[pl.ds(s * RPW, RPW)])
    pltpu.sync_copy(support_hbm.at[pl.ds(s * RPW, RPW)],
                    support_s.at[pl.ds(s * RPW, RPW)])
    plsc.subcore_barrier()

    def gather(k, rows, sem):
        pltpu.async_copy(support_s.at[src_v.at[k]], rows, sem)

    def wait_gather(k, rows, sem):
        pltpu.make_async_copy(support_s.at[src_v.at[k]], rows, sem).wait()

    def scale(k, rows, msgs):
        @plsc.parallel_loop(0, CHUNK // L, step=1, unroll=2)
        def group_body(g):
            w16 = w_v[k, pl.ds(g * L, L)]
            for i in range(L):
                e = g * L + i
                wspl = w16[i]
                for h in range(D_HALF // L):
                    word = rows[e, pl.ds(h * L, L)]
                    lo = lax.bitcast_convert_type(word << 16, jnp.float32)
                    hi = lax.bitcast_convert_type(
                        word & jnp.int32(-65536), jnp.float32)
                    msgs[e, pl.ds(h * L, L)] = lo * wspl
                    msgs[e, pl.ds(D_HALF + h * L, L)] = hi * wspl

    def scatter(k, msgs, sem):
        pltpu.async_copy(msgs, accum.at[dst_v.at[k]], sem, add=True)

    def wait_scatter(k, msgs, sem):
        pltpu.make_async_copy(msgs, accum.at[dst_v.at[k]], sem).wait()

    gather(0, rows0, gsem0)
    gather(1, rows1, gsem1)

    wait_gather(0, rows0, gsem0)
    scale(0, rows0, msgs0)
    scatter(0, msgs0, ssem0)
    gather(2, rows0, gsem0)
    wait_gather(1, rows1, gsem1)
    scale(1, rows1, msgs1)
    scatter(1, msgs1, ssem1)
    gather(3, rows1, gsem1)

    def chunk_body(k, _):
        k2 = 2 * k
        wait_gather(k2, rows0, gsem0)
        wait_scatter(k2 - 2, msgs0, ssem0)
        scale(k2, rows0, msgs0)
        scatter(k2, msgs0, ssem0)
        gather(k2 + 2, rows0, gsem0)
        wait_gather(k2 + 1, rows1, gsem1)
        wait_scatter(k2 - 1, msgs1, ssem1)
        scale(k2 + 1, rows1, msgs1)
        scatter(k2 + 1, msgs1, ssem1)
        gather(k2 + 3, rows1, gsem1)
        return ()

    lax.fori_loop(1, CHUNKS_PW // 2 - 1, chunk_body, ())
    k_last = CHUNKS_PW - 2
    wait_gather(k_last, rows0, gsem0)
    wait_scatter(k_last - 2, msgs0, ssem0)
    scale(k_last, rows0, msgs0)
    scatter(k_last, msgs0, ssem0)
    wait_gather(k_last + 1, rows1, gsem1)
    wait_scatter(k_last - 1, msgs1, ssem1)
    scale(k_last + 1, rows1, msgs1)
    scatter(k_last + 1, msgs1, ssem1)
    wait_scatter(k_last, msgs0, ssem0)
    wait_scatter(k_last + 1, msgs1, ssem1)

    plsc.subcore_barrier()
    pltpu.sync_copy(accum.at[pl.ds(s * RPW, RPW)],
                    out_hbm.at[c, pl.ds(s * RPW, RPW)])


def kernel(x, edge_index, edge_weight, W1, b1, W2, b2):
    x_pad = jnp.concatenate(
        [x, jnp.zeros((N_PAD - N_NODES, x.shape[1]), jnp.float32)])
    support = pl.pallas_call(
        _matmul_body,
        out_shape=jax.ShapeDtypeStruct((N_PAD, D_HALF), jnp.int32),
    )(x_pad, W2)

    n_edges = edge_weight.shape[0]
    pad = E_PAD - n_edges
    src = jnp.concatenate([edge_index[0], jnp.zeros((pad,), jnp.int32)])
    src = src.reshape(NW, CHUNKS_PW, CHUNK)
    dst = jnp.concatenate([edge_index[1], jnp.zeros((pad,), jnp.int32)])
    dst = dst.reshape(NW, CHUNKS_PW, CHUNK)
    w = jnp.concatenate([edge_weight, jnp.zeros((pad,), jnp.float32)])
    w = w.reshape(NW, CHUNKS_PW, CHUNK)
    zeros = jnp.zeros((N_PAD, D_OUT), jnp.float32)

    partials = _edge_scatter(support, src, dst, w, zeros)[:, :N_NODES, :]

    b2r = b2.reshape(1, D_OUT)
    out = pl.pallas_call(
        _finish_body,
        out_shape=jax.ShapeDtypeStruct((N_NODES, D_OUT), jnp.float32),
    )(partials, b2r)
    return out


# speedup vs baseline: 1.2418x; 1.2418x over previous
import functools

import jax
import jax.numpy as jnp
from jax import lax
from jax.experimental import pallas as pl
from jax.experimental.pallas import tpu as pltpu
from jax.experimental.pallas import tpu_sc as plsc

N_NODES = 10000
D_OUT = 64
D_HALF = D_OUT // 2
NC = 2
NS = 16
L = 16
NW = NC * NS
CHUNK = 80
CHUNKS_PW = 125
EPW = CHUNK * CHUNKS_PW
RPW = N_NODES // NS


def _matmul_body(x_ref, w_ref, o_ref):
    z = jnp.dot(x_ref[...], w_ref[...], preferred_element_type=jnp.float32)
    zb = z.astype(jnp.bfloat16)
    lo = lax.bitcast_convert_type(zb[:, :D_HALF], jnp.uint16
                                  ).astype(jnp.uint32)
    hi = lax.bitcast_convert_type(zb[:, D_HALF:], jnp.uint16
                                  ).astype(jnp.uint32)
    o_ref[...] = lax.bitcast_convert_type(lo | (hi << 16), jnp.int32)


def _finish_body(p_ref, b_ref, o_ref):
    z = p_ref[0] + p_ref[1] + b_ref[...]
    m = jnp.max(z, axis=1, keepdims=True)
    zz = z - m
    lse = jnp.log(jnp.sum(jnp.exp(zz), axis=1, keepdims=True))
    o_ref[...] = zz - lse


_mesh = plsc.VectorSubcoreMesh(core_axis_name="c", subcore_axis_name="s",
                               num_cores=NC, num_subcores=NS)


@functools.partial(
    pl.kernel,
    out_type=jax.ShapeDtypeStruct((NC, N_NODES, D_OUT), jnp.float32),
    mesh=_mesh,
    scratch_types=[
        pltpu.VMEM((CHUNKS_PW, CHUNK), jnp.int32),
        pltpu.VMEM((CHUNKS_PW, CHUNK), jnp.int32),
        pltpu.VMEM((CHUNKS_PW, CHUNK), jnp.float32),
        pltpu.VMEM((CHUNK, D_HALF), jnp.int32),
        pltpu.VMEM((CHUNK, D_HALF), jnp.int32),
        pltpu.VMEM((CHUNK, D_OUT), jnp.float32),
        pltpu.VMEM((CHUNK, D_OUT), jnp.float32),
        pltpu.VMEM_SHARED((N_NODES, D_OUT), jnp.float32),
        pltpu.VMEM_SHARED((N_NODES, D_HALF), jnp.int32),
        pltpu.SemaphoreType.DMA,
        pltpu.SemaphoreType.DMA,
        pltpu.SemaphoreType.DMA,
        pltpu.SemaphoreType.DMA,
    ],
    compiler_params=pltpu.CompilerParams(use_tc_tiling_on_sc=False),
)
def _edge_scatter(support_hbm, src_hbm, dst_hbm, w_hbm, out_hbm,
                  src_v, dst_v, w_v, rows0, rows1, msgs0, msgs1, accum,
                  support_s, gsem0, gsem1, ssem0, ssem1):
    c = lax.axis_index("c")
    s = lax.axis_index("s")
    wid = s * NC + c

    pltpu.sync_copy(src_hbm.at[wid], src_v)
    pltpu.sync_copy(dst_hbm.at[wid], dst_v)
    pltpu.sync_copy(w_hbm.at[wid], w_v)
    pltpu.sync_copy(support_hbm.at[pl.ds(s * RPW, RPW)],
                    support_s.at[pl.ds(s * RPW, RPW)])

    @plsc.parallel_loop(0, CHUNK * D_OUT // L, step=1, unroll=4)
    def zero_body(i):
        msgs0[pl.ds(i // (D_OUT // L), 1), pl.ds((i % (D_OUT // L)) * L, L)] \
            = jnp.zeros((1, L), jnp.float32)

    for i in range(7):
        pltpu.sync_copy(msgs0, accum.at[pl.ds(s * RPW + i * CHUNK, CHUNK)])
    pltpu.sync_copy(msgs0.at[pl.ds(0, RPW - 7 * CHUNK)],
                    accum.at[pl.ds(s * RPW + 7 * CHUNK, RPW - 7 * CHUNK)])
    plsc.subcore_barrier()

    def gather(k, rows, sem):
        pltpu.async_copy(support_s.at[src_v.at[k]], rows, sem)

    def wait_gather(k, rows, sem):
        pltpu.make_async_copy(support_s.at[src_v.at[k]], rows, sem).wait()

    def scale(k, rows, msgs):
        @plsc.parallel_loop(0, CHUNK // L, step=1, unroll=2)
        def group_body(g):
            w16 = w_v[k, pl.ds(g * L, L)]
            for i in range(L):
                e = g * L + i
                wspl = w16[i]
                for h in range(D_HALF // L):
                    word = rows[e, pl.ds(h * L, L)]
                    lo = lax.bitcast_convert_type(word << 16, jnp.float32)
                    hi = lax.bitcast_convert_type(
                        word & jnp.int32(-65536), jnp.float32)
                    msgs[e, pl.ds(h * L, L)] = lo * wspl
                    msgs[e, pl.ds(D_HALF + h * L, L)] = hi * wspl

    def scatter(k, msgs, sem):
        pltpu.async_copy(msgs, accum.at[dst_v.at[k]], sem, add=True)

    def wait_scatter(k, msgs, sem):
        pltpu.make_async_copy(msgs, accum.at[dst_v.at[k]], sem).wait()

    gather(0, rows0, gsem0)
    gather(1, rows1, gsem1)

    wait_gather(0, rows0, gsem0)
    scale(0, rows0, msgs0)
    scatter(0, msgs0, ssem0)
    gather(2, rows0, gsem0)
    wait_gather(1, rows1, gsem1)
    scale(1, rows1, msgs1)
    scatter(1, msgs1, ssem1)
    gather(3, rows1, gsem1)

    def chunk_body(k, _):
        k2 = 2 * k
        wait_gather(k2, rows0, gsem0)
        wait_scatter(k2 - 2, msgs0, ssem0)
        scale(k2, rows0, msgs0)
        scatter(k2, msgs0, ssem0)
        gather(k2 + 2, rows0, gsem0)
        wait_gather(k2 + 1, rows1, gsem1)
        wait_scatter(k2 - 1, msgs1, ssem1)
        scale(k2 + 1, rows1, msgs1)
        scatter(k2 + 1, msgs1, ssem1)
        gather(k2 + 3, rows1, gsem1)
        return ()

    lax.fori_loop(1, (CHUNKS_PW - 3) // 2, chunk_body, ())
    k_last = CHUNKS_PW - 3
    wait_gather(k_last, rows0, gsem0)
    wait_scatter(k_last - 2, msgs0, ssem0)
    scale(k_last, rows0, msgs0)
    scatter(k_last, msgs0, ssem0)
    gather(k_last + 2, rows0, gsem0)
    wait_gather(k_last + 1, rows1, gsem1)
    wait_scatter(k_last - 1, msgs1, ssem1)
    scale(k_last + 1, rows1, msgs1)
    scatter(k_last + 1, msgs1, ssem1)
    wait_gather(k_last + 2, rows0, gsem0)
    wait_scatter(k_last, msgs0, ssem0)
    scale(k_last + 2, rows0, msgs0)
    scatter(k_last + 2, msgs0, ssem0)
    wait_scatter(k_last + 1, msgs1, ssem1)
    wait_scatter(k_last + 2, msgs0, ssem0)

    plsc.subcore_barrier()
    pltpu.sync_copy(accum.at[pl.ds(s * RPW, RPW)],
                    out_hbm.at[c, pl.ds(s * RPW, RPW)])


def kernel(x, edge_index, edge_weight, W1, b1, W2, b2):
    support = pl.pallas_call(
        _matmul_body,
        out_shape=jax.ShapeDtypeStruct((N_NODES, D_HALF), jnp.int32),
    )(x, W2)

    src = edge_index[0].reshape(NW, CHUNKS_PW, CHUNK)
    dst = edge_index[1].reshape(NW, CHUNKS_PW, CHUNK)
    w = edge_weight.reshape(NW, CHUNKS_PW, CHUNK)

    partials = _edge_scatter(support, src, dst, w)

    b2r = b2.reshape(1, D_OUT)
    out = pl.pallas_call(
        _finish_body,
        out_shape=jax.ShapeDtypeStruct((N_NODES, D_OUT), jnp.float32),
    )(partials, b2r)
    return out
